# baseline (speedup 1.0000x reference)
import functools

import jax
import jax.numpy as jnp
from jax import lax
from jax.experimental import pallas as pl
from jax.experimental.pallas import tpu as pltpu
from jax.experimental.pallas import tpu_sc as plsc


@functools.lru_cache(maxsize=None)
def _make_gather(V, D, R, S):
    info = plsc.get_sparse_core_info()
    NC, NS, L = info.num_cores, info.num_subcores, info.num_lanes
    NW = NC * NS
    assert R % (NW * 128) == 0 and D % 8 == 0
    RB = R // 128
    assert RB == NW
    mesh = plsc.VectorSubcoreMesh(core_axis_name="c", subcore_axis_name="s")

    @functools.partial(
        pl.kernel,
        mesh=mesh,
        out_type=jax.ShapeDtypeStruct((S, D // 8, RB, 8, 128), jnp.float32),
        scratch_types=[
            pltpu.VMEM((S, 128), jnp.int32),
            pltpu.VMEM((2, 128, D), jnp.float32),
            pltpu.VMEM((2, D // 8, 8, 128), jnp.float32),
            [pltpu.SemaphoreType.DMA] * 2,
            [pltpu.SemaphoreType.DMA] * 2,
        ],
        compiler_params=pltpu.CompilerParams(
            use_tc_tiling_on_sc=False, needs_layout_passes=False
        ),
    )
    def gather_kernel(idx_hbm, table_hbm, out_hbm, idx_v, rows_v, tile_v,
                      gsems, osems):
        wid = lax.axis_index("s") * NC + lax.axis_index("c")
        pltpu.sync_copy(idx_hbm.at[:, pl.ds(wid * 128, 128)], idx_v)

        lane = lax.iota(jnp.int32, L)

        def transpose_block(b):
            def fbody(f, carry):
                a = f // 8
                fa = f % 8
                col = jnp.full((L,), 0, jnp.int32) + f
                for g in range(128 // L):
                    row = lane + (g * L)
                    vals = plsc.load_gather(rows_v.at[b], [row, col])
                    tile_v[b, a, fa, pl.ds(g * L, L)] = vals
                return carry

            lax.fori_loop(0, D, fbody, 0)

        def start_gather(s, b):
            pltpu.async_copy(
                table_hbm.at[idx_v.at[s]], rows_v.at[b], gsems[b]
            )

        def wait_gather(s, b):
            pltpu.make_async_copy(
                table_hbm.at[idx_v.at[s]], rows_v.at[b], gsems[b]
            ).wait()

        def start_out(s, b):
            pltpu.async_copy(tile_v.at[b], out_hbm.at[s, :, wid], osems[b])

        def wait_out(s, b):
            pltpu.make_async_copy(
                tile_v.at[b], out_hbm.at[s, :, wid], osems[b]
            ).wait()

        start_gather(0, 0)
        start_gather(1, 1)

        def body(g, carry):
            s0 = g * 2
            for b in range(2):
                s = s0 + b
                wait_gather(s, b)

                @pl.when(g > 0)
                def _():
                    wait_out(s - 2, b)

                transpose_block(b)
                start_out(s, b)

                @pl.when(g < S // 2 - 1)
                def _():
                    start_gather(s + 2, b)

            return carry

        lax.fori_loop(0, S // 2, body, 0)
        wait_out(S - 2, 0)
        wait_out(S - 1, 1)

    return gather_kernel


def kernel(inputs, embeddings):
    V, D = embeddings.shape
    R, S = inputs.shape
    idx_t = inputs.T.astype(jnp.int32)
    out5 = _make_gather(V, D, R, S)(idx_t, embeddings)
    return out5.transpose(2, 4, 0, 1, 3).reshape(R, S, D)
